# baseline (speedup 1.0000x reference)
import functools

import jax
import jax.numpy as jnp
from jax import lax
from jax.experimental import pallas as pl
from jax.experimental.pallas import tpu as pltpu
from jax.experimental.pallas import tpu_sc as plsc

N_CORES = 2
N_SUBCORES = 16
NW = N_CORES * N_SUBCORES
L = 16


def _build_sc_kernel(n_sites, npix, chunk, groups, n_chunks, shift_w, mask_w):
    mesh = plsc.VectorSubcoreMesh(
        core_axis_name="c", subcore_axis_name="s",
        num_cores=N_CORES, num_subcores=N_SUBCORES)
    per_w = npix // NW

    @functools.partial(
        pl.kernel,
        out_type=jax.ShapeDtypeStruct((npix * 3,), jnp.float32),
        mesh=mesh,
        scratch_types=[
            pltpu.VMEM((n_sites * 5,), jnp.float32),
            pltpu.VMEM((chunk,), jnp.int32),
            pltpu.VMEM((chunk,), jnp.int32),
            [pltpu.VMEM((chunk,), jnp.float32) for _ in range(3)],
            pltpu.VMEM((3 * L,), jnp.float32),
        ],
        compiler_params=pltpu.CompilerParams(use_tc_tiling_on_sc=False,
                                             needs_layout_passes=False),
    )
    def sad_sc(sites_hbm, c0_hbm, c1_hbm, par_hbm, out_hbm,
               table_v, c0_v, c1_v, out_v, par_v):
        wid = lax.axis_index("s") * N_CORES + lax.axis_index("c")
        pltpu.sync_copy(sites_hbm, table_v)
        pltpu.sync_copy(par_hbm, par_v)
        inv_w = par_v[pl.ds(0, L)]
        inv_h = par_v[pl.ds(L, L)]
        scale = par_v[pl.ds(2 * L, L)]
        iota = lax.iota(jnp.int32, L)
        base_w = wid * per_w

        def chunk_body(ci, carry):
            base = base_w + ci * chunk
            pltpu.sync_copy(c0_hbm.at[pl.ds(base, chunk)], c0_v)
            pltpu.sync_copy(c1_hbm.at[pl.ds(base, chunk)], c1_v)

            row0 = lax.shift_right_logical(base, shift_w)

            @plsc.parallel_loop(0, groups, 1, unroll=8)
            def grp(g):
                off = g * L
                cc = lax.shift_right_logical(g, 6)
                rem = g & 63
                r = lax.shift_right_logical(rem, 3)
                l0 = (rem & 7) * L
                qoff = r * (mask_w + 1) + cc * 128 + l0
                idx0 = c0_v[pl.ds(qoff, L)]
                idx1 = c1_v[pl.ds(qoff, L)]
                x0 = plsc.load_gather(table_v, [idx0])
                y0 = plsc.load_gather(table_v, [idx0 + 1])
                x1 = plsc.load_gather(table_v, [idx1])
                y1 = plsc.load_gather(table_v, [idx1 + 1])
                xi = cc * 128 + l0 + iota
                px = (xi.astype(jnp.float32) + 0.5) * inv_w
                yi = jnp.full((L,), row0 + r, jnp.int32)
                py = (yi.astype(jnp.float32) + 0.5) * inv_h
                dx0 = px - x0
                dy0 = py - y0
                dx1 = px - x1
                dy1 = py - y1
                d0 = dx0 * dx0 + dy0 * dy0
                d1 = dx1 * dx1 + dy1 * dy1
                t = (d1 - d0) * scale
                w = 1.0 / (1.0 + jnp.exp(-t))
                for c in range(3):
                    a = plsc.load_gather(table_v, [idx0 + (2 + c)])
                    b = plsc.load_gather(table_v, [idx1 + (2 + c)])
                    out_v[c][pl.ds(off, L)] = b + w * (a - b)

            for c in range(3):
                pltpu.sync_copy(out_v[c], out_hbm.at[pl.ds(c * npix + base, chunk)])
            return carry

        lax.fori_loop(0, n_chunks, chunk_body, 0)

    return sad_sc


def kernel(sites, cand0, cand1, width, height, inv_scale_sq):
    height_s, width_s = cand0.shape
    n_sites = sites.shape[0]
    npix = height_s * width_s
    assert width_s & (width_s - 1) == 0, "width must be a power of two"
    shift_w = width_s.bit_length() - 1
    mask_w = width_s - 1
    per_w = npix // NW
    chunk = min(8192, per_w)
    groups = chunk // L
    n_chunks = per_w // chunk

    width_f = jnp.asarray(width, dtype=jnp.float32)
    height_f = jnp.asarray(height, dtype=jnp.float32)
    scale_f = jnp.asarray(inv_scale_sq, dtype=jnp.float32)
    params = jnp.concatenate([
        jnp.broadcast_to(1.0 / width_f, (L,)),
        jnp.broadcast_to(1.0 / height_f, (L,)),
        jnp.broadcast_to(scale_f, (L,)),
    ]).astype(jnp.float32)

    sad_sc = _build_sc_kernel(n_sites, npix, chunk, groups, n_chunks,
                              shift_w, mask_w)
    out_flat = sad_sc(sites.reshape(n_sites * 5), cand0.reshape(npix) * 5,
                      cand1.reshape(npix) * 5, params)
    out5 = out_flat.reshape(3, height_s // 8, width_s // 128, 8, 128)
    return out5.transpose(1, 3, 2, 4, 0).reshape(height_s, width_s, 3)


# speedup vs baseline: 1.0858x; 1.0858x over previous
import functools

import jax
import jax.numpy as jnp
from jax import lax
from jax.experimental import pallas as pl
from jax.experimental.pallas import tpu as pltpu
from jax.experimental.pallas import tpu_sc as plsc

N_CORES = 2
N_SUBCORES = 16
NW = N_CORES * N_SUBCORES
L = 16


def _build_sc_kernel(n_sites, npix, chunk, groups, n_chunks, shift_w, mask_w):
    mesh = plsc.VectorSubcoreMesh(
        core_axis_name="c", subcore_axis_name="s",
        num_cores=N_CORES, num_subcores=N_SUBCORES)
    per_w = npix // NW

    @functools.partial(
        pl.kernel,
        out_type=jax.ShapeDtypeStruct((npix * 3,), jnp.float32),
        mesh=mesh,
        scratch_types=[
            pltpu.VMEM((n_sites * 5,), jnp.float32),
            pltpu.VMEM((chunk,), jnp.int32),
            pltpu.VMEM((chunk,), jnp.int32),
            [pltpu.VMEM((chunk,), jnp.float32) for _ in range(3)],
            pltpu.VMEM((3 * L,), jnp.float32),
        ],
        compiler_params=pltpu.CompilerParams(use_tc_tiling_on_sc=False,
                                             needs_layout_passes=False),
    )
    def sad_sc(sites_hbm, c0_hbm, c1_hbm, par_hbm, out_hbm,
               table_v, c0_v, c1_v, out_v, par_v):
        wid = lax.axis_index("s") * N_CORES + lax.axis_index("c")
        pltpu.sync_copy(sites_hbm, table_v)
        pltpu.sync_copy(par_hbm, par_v)
        inv_w = par_v[pl.ds(0, L)]
        inv_h = par_v[pl.ds(L, L)]
        scale = par_v[pl.ds(2 * L, L)]
        iota = lax.iota(jnp.int32, L)
        base_w = wid * per_w

        def chunk_body(ci, carry):
            base = base_w + ci * chunk
            pltpu.sync_copy(c0_hbm.at[pl.ds(base, chunk)], c0_v)
            pltpu.sync_copy(c1_hbm.at[pl.ds(base, chunk)], c1_v)

            row0 = lax.shift_right_logical(base, shift_w)

            @plsc.parallel_loop(0, groups, 1, unroll=4)
            def grp(g):
                off = g * L
                cc = lax.shift_right_logical(g, 6)
                rem = g & 63
                r = lax.shift_right_logical(rem, 3)
                l0 = (rem & 7) * L
                qoff = r * (mask_w + 1) + cc * 128 + l0
                idx0 = c0_v[pl.ds(qoff, L)]
                idx1 = c1_v[pl.ds(qoff, L)]
                x0 = plsc.load_gather(table_v, [idx0])
                y0 = plsc.load_gather(table_v, [idx0 + 1])
                x1 = plsc.load_gather(table_v, [idx1])
                y1 = plsc.load_gather(table_v, [idx1 + 1])
                xi = cc * 128 + l0 + iota
                px = (xi.astype(jnp.float32) + 0.5) * inv_w
                yi = jnp.full((L,), row0 + r, jnp.int32)
                py = (yi.astype(jnp.float32) + 0.5) * inv_h
                dx0 = px - x0
                dy0 = py - y0
                dx1 = px - x1
                dy1 = py - y1
                d0 = dx0 * dx0 + dy0 * dy0
                d1 = dx1 * dx1 + dy1 * dy1
                t = (d1 - d0) * scale
                w = 1.0 / (1.0 + jnp.exp(-t))
                for c in range(3):
                    a = plsc.load_gather(table_v, [idx0 + (2 + c)])
                    b = plsc.load_gather(table_v, [idx1 + (2 + c)])
                    out_v[c][pl.ds(off, L)] = b + w * (a - b)

            for c in range(3):
                pltpu.sync_copy(out_v[c], out_hbm.at[pl.ds(c * npix + base, chunk)])
            return carry

        lax.fori_loop(0, n_chunks, chunk_body, 0)

    return sad_sc


def kernel(sites, cand0, cand1, width, height, inv_scale_sq):
    height_s, width_s = cand0.shape
    n_sites = sites.shape[0]
    npix = height_s * width_s
    assert width_s & (width_s - 1) == 0, "width must be a power of two"
    shift_w = width_s.bit_length() - 1
    mask_w = width_s - 1
    per_w = npix // NW
    chunk = min(8192, per_w)
    groups = chunk // L
    n_chunks = per_w // chunk

    width_f = jnp.asarray(width, dtype=jnp.float32)
    height_f = jnp.asarray(height, dtype=jnp.float32)
    scale_f = jnp.asarray(inv_scale_sq, dtype=jnp.float32)
    params = jnp.concatenate([
        jnp.broadcast_to(1.0 / width_f, (L,)),
        jnp.broadcast_to(1.0 / height_f, (L,)),
        jnp.broadcast_to(scale_f, (L,)),
    ]).astype(jnp.float32)

    sad_sc = _build_sc_kernel(n_sites, npix, chunk, groups, n_chunks,
                              shift_w, mask_w)
    out_flat = sad_sc(sites.reshape(n_sites * 5), cand0.reshape(npix) * 5,
                      cand1.reshape(npix) * 5, params)
    out5 = out_flat.reshape(3, height_s // 8, width_s // 128, 8, 128)
    return out5.transpose(1, 3, 2, 4, 0).reshape(height_s, width_s, 3)


# speedup vs baseline: 1.1091x; 1.0215x over previous
import functools

import jax
import jax.numpy as jnp
from jax import lax
from jax.experimental import pallas as pl
from jax.experimental.pallas import tpu as pltpu
from jax.experimental.pallas import tpu_sc as plsc

N_CORES = 2
N_SUBCORES = 16
NW = N_CORES * N_SUBCORES
L = 16


def _build_sc_kernel(n_sites, npix, chunk, groups, n_chunks, shift_w, mask_w):
    mesh = plsc.VectorSubcoreMesh(
        core_axis_name="c", subcore_axis_name="s",
        num_cores=N_CORES, num_subcores=N_SUBCORES)
    per_w = npix // NW

    @functools.partial(
        pl.kernel,
        out_type=jax.ShapeDtypeStruct((npix * 3,), jnp.float32),
        mesh=mesh,
        scratch_types=[
            pltpu.VMEM((n_sites * 5,), jnp.float32),
            pltpu.VMEM((chunk,), jnp.int32),
            pltpu.VMEM((chunk,), jnp.int32),
            [pltpu.VMEM((chunk,), jnp.float32) for _ in range(3)],
            pltpu.VMEM((3 * L,), jnp.float32),
        ],
        compiler_params=pltpu.CompilerParams(use_tc_tiling_on_sc=False,
                                             needs_layout_passes=False),
    )
    def sad_sc(sites_hbm, c0_hbm, c1_hbm, par_hbm, out_hbm,
               table_v, c0_v, c1_v, out_v, par_v):
        wid = lax.axis_index("s") * N_CORES + lax.axis_index("c")
        pltpu.sync_copy(sites_hbm, table_v)
        pltpu.sync_copy(par_hbm, par_v)
        inv_w = par_v[pl.ds(0, L)]
        inv_h = par_v[pl.ds(L, L)]
        scale = par_v[pl.ds(2 * L, L)]
        iota = lax.iota(jnp.int32, L)
        base_w = wid * per_w

        def chunk_body(ci, carry):
            base = base_w + ci * chunk
            pltpu.sync_copy(c0_hbm.at[pl.ds(base, chunk)], c0_v)
            pltpu.sync_copy(c1_hbm.at[pl.ds(base, chunk)], c1_v)

            row0 = lax.shift_right_logical(base, shift_w)

            @plsc.parallel_loop(0, groups, 1, unroll=4)
            def grp(g):
                off = g * L
                cc = lax.shift_right_logical(g, 6)
                rem = g & 63
                r = lax.shift_right_logical(rem, 3)
                l0 = (rem & 7) * L
                qoff = r * (mask_w + 1) + cc * 128 + l0
                idx0 = c0_v[pl.ds(qoff, L)] * 5
                idx1 = c1_v[pl.ds(qoff, L)] * 5
                x0 = plsc.load_gather(table_v, [idx0])
                y0 = plsc.load_gather(table_v, [idx0 + 1])
                x1 = plsc.load_gather(table_v, [idx1])
                y1 = plsc.load_gather(table_v, [idx1 + 1])
                xi = cc * 128 + l0 + iota
                px = (xi.astype(jnp.float32) + 0.5) * inv_w
                yi = jnp.full((L,), row0 + r, jnp.int32)
                py = (yi.astype(jnp.float32) + 0.5) * inv_h
                dx0 = px - x0
                dy0 = py - y0
                dx1 = px - x1
                dy1 = py - y1
                d0 = dx0 * dx0 + dy0 * dy0
                d1 = dx1 * dx1 + dy1 * dy1
                t = (d1 - d0) * scale
                w = 1.0 / (1.0 + jnp.exp(-t))
                for c in range(3):
                    a = plsc.load_gather(table_v, [idx0 + (2 + c)])
                    b = plsc.load_gather(table_v, [idx1 + (2 + c)])
                    out_v[c][pl.ds(off, L)] = b + w * (a - b)

            for c in range(3):
                pltpu.sync_copy(out_v[c], out_hbm.at[pl.ds(c * npix + base, chunk)])
            return carry

        lax.fori_loop(0, n_chunks, chunk_body, 0)

    return sad_sc


def kernel(sites, cand0, cand1, width, height, inv_scale_sq):
    height_s, width_s = cand0.shape
    n_sites = sites.shape[0]
    npix = height_s * width_s
    assert width_s & (width_s - 1) == 0, "width must be a power of two"
    shift_w = width_s.bit_length() - 1
    mask_w = width_s - 1
    per_w = npix // NW
    chunk = min(8192, per_w)
    groups = chunk // L
    n_chunks = per_w // chunk

    width_f = jnp.asarray(width, dtype=jnp.float32)
    height_f = jnp.asarray(height, dtype=jnp.float32)
    scale_f = jnp.asarray(inv_scale_sq, dtype=jnp.float32)
    params = jnp.concatenate([
        jnp.broadcast_to(1.0 / width_f, (L,)),
        jnp.broadcast_to(1.0 / height_f, (L,)),
        jnp.broadcast_to(scale_f, (L,)),
    ]).astype(jnp.float32)

    sad_sc = _build_sc_kernel(n_sites, npix, chunk, groups, n_chunks,
                              shift_w, mask_w)
    out_flat = sad_sc(sites.reshape(n_sites * 5), cand0.reshape(npix),
                      cand1.reshape(npix), params)
    out5 = out_flat.reshape(3, height_s // 8, width_s // 128, 8, 128)
    return out5.transpose(1, 3, 2, 4, 0).reshape(height_s, width_s, 3)


# speedup vs baseline: 1.4729x; 1.3281x over previous
import functools

import jax
import jax.numpy as jnp
from jax import lax
from jax.experimental import pallas as pl
from jax.experimental.pallas import tpu as pltpu
from jax.experimental.pallas import tpu_sc as plsc

N_CORES = 2
N_SUBCORES = 16
NW = N_CORES * N_SUBCORES
L = 16


def _build_sc_kernel(n_sites, npix, chunk, groups, n_chunks, shift_w, mask_w):
    mesh = plsc.VectorSubcoreMesh(
        core_axis_name="c", subcore_axis_name="s",
        num_cores=N_CORES, num_subcores=N_SUBCORES)
    per_w = npix // NW

    @functools.partial(
        pl.kernel,
        out_type=jax.ShapeDtypeStruct((npix * 3,), jnp.float32),
        mesh=mesh,
        scratch_types=[
            pltpu.VMEM((n_sites * 5,), jnp.float32),
            pltpu.VMEM((chunk,), jnp.int32),
            pltpu.VMEM((chunk,), jnp.int32),
            [pltpu.VMEM((chunk,), jnp.float32) for _ in range(3)],
            pltpu.VMEM((3 * L,), jnp.float32),
        ],
        compiler_params=pltpu.CompilerParams(use_tc_tiling_on_sc=False,
                                             needs_layout_passes=False),
    )
    def sad_sc(sites_hbm, c0_hbm, c1_hbm, par_hbm, out_hbm,
               table_v, c0_v, c1_v, out_v, par_v):
        wid = lax.axis_index("s") * N_CORES + lax.axis_index("c")
        pltpu.sync_copy(sites_hbm, table_v)
        pltpu.sync_copy(par_hbm, par_v)
        inv_w = par_v[pl.ds(0, L)]
        inv_h = par_v[pl.ds(L, L)]
        scale = par_v[pl.ds(2 * L, L)]
        iota = lax.iota(jnp.int32, L)
        base_w = wid * per_w

        def chunk_body(ci, carry):
            base = base_w + ci * chunk
            pltpu.sync_copy(c0_hbm.at[pl.ds(base, chunk)], c0_v)
            pltpu.sync_copy(c1_hbm.at[pl.ds(base, chunk)], c1_v)

            row0 = lax.shift_right_logical(base, shift_w)

            @plsc.parallel_loop(0, groups, 1, unroll=4)
            def grp(g):
                off = g * L
                cc = lax.shift_right_logical(g, 6)
                rem = g & 63
                r = lax.shift_right_logical(rem, 3)
                l0 = (rem & 7) * L
                idx0 = c0_v[pl.ds(off, L)]
                idx1 = c1_v[pl.ds(off, L)]
                x0 = plsc.load_gather(table_v, [idx0])
                y0 = plsc.load_gather(table_v, [idx0 + n_sites])
                x1 = plsc.load_gather(table_v, [idx1])
                y1 = plsc.load_gather(table_v, [idx1 + n_sites])
                xi = cc * 128 + l0 + iota
                px = (xi.astype(jnp.float32) + 0.5) * inv_w
                yi = jnp.full((L,), row0 + r, jnp.int32)
                py = (yi.astype(jnp.float32) + 0.5) * inv_h
                dx0 = px - x0
                dy0 = py - y0
                dx1 = px - x1
                dy1 = py - y1
                d0 = dx0 * dx0 + dy0 * dy0
                d1 = dx1 * dx1 + dy1 * dy1
                t = (d1 - d0) * scale
                w = 1.0 / (1.0 + jnp.exp(-t))
                for c in range(3):
                    a = plsc.load_gather(table_v, [idx0 + (2 + c) * n_sites])
                    b = plsc.load_gather(table_v, [idx1 + (2 + c) * n_sites])
                    out_v[c][pl.ds(off, L)] = b + w * (a - b)

            for c in range(3):
                pltpu.sync_copy(out_v[c], out_hbm.at[pl.ds(c * npix + base, chunk)])
            return carry

        lax.fori_loop(0, n_chunks, chunk_body, 0)

    return sad_sc


def kernel(sites, cand0, cand1, width, height, inv_scale_sq):
    height_s, width_s = cand0.shape
    n_sites = sites.shape[0]
    npix = height_s * width_s
    assert width_s & (width_s - 1) == 0, "width must be a power of two"
    shift_w = width_s.bit_length() - 1
    mask_w = width_s - 1
    per_w = npix // NW
    chunk = min(8192, per_w)
    groups = chunk // L
    n_chunks = per_w // chunk

    width_f = jnp.asarray(width, dtype=jnp.float32)
    height_f = jnp.asarray(height, dtype=jnp.float32)
    scale_f = jnp.asarray(inv_scale_sq, dtype=jnp.float32)
    params = jnp.concatenate([
        jnp.broadcast_to(1.0 / width_f, (L,)),
        jnp.broadcast_to(1.0 / height_f, (L,)),
        jnp.broadcast_to(scale_f, (L,)),
    ]).astype(jnp.float32)

    sad_sc = _build_sc_kernel(n_sites, npix, chunk, groups, n_chunks,
                              shift_w, mask_w)
    def tile_order(c):
        c4 = c.reshape(height_s // 8, 8, width_s // 128, 128)
        return c4.transpose(0, 2, 1, 3).reshape(npix)

    sites_cols = sites.T.reshape(n_sites * 5)
    out_flat = sad_sc(sites_cols, tile_order(cand0), tile_order(cand1),
                      params)
    out5 = out_flat.reshape(3, height_s // 8, width_s // 128, 8, 128)
    return out5.transpose(1, 3, 2, 4, 0).reshape(height_s, width_s, 3)


# speedup vs baseline: 1.6684x; 1.1327x over previous
import functools

import jax
import jax.numpy as jnp
from jax import lax
from jax.experimental import pallas as pl
from jax.experimental.pallas import tpu as pltpu
from jax.experimental.pallas import tpu_sc as plsc

N_CORES = 2
N_SUBCORES = 16
NW = N_CORES * N_SUBCORES
L = 16
CHUNK = 4096


def _build_sc_kernel(n_sites, npix, width_s):
    mesh = plsc.VectorSubcoreMesh(
        core_axis_name="c", subcore_axis_name="s",
        num_cores=N_CORES, num_subcores=N_SUBCORES)
    per_w = npix // NW
    n_chunks = per_w // CHUNK
    groups = CHUNK // L
    tiles_w = width_s // 128

    @functools.partial(
        pl.kernel,
        out_type=jax.ShapeDtypeStruct((npix * 3,), jnp.float32),
        mesh=mesh,
        scratch_types=[
            pltpu.VMEM((n_sites * 5,), jnp.float32),
            [pltpu.VMEM((CHUNK,), jnp.int32) for _ in range(2)],
            [pltpu.VMEM((CHUNK,), jnp.int32) for _ in range(2)],
            [[pltpu.VMEM((CHUNK,), jnp.float32) for _ in range(3)]
             for _ in range(2)],
            pltpu.VMEM((3 * L,), jnp.float32),
            pltpu.SemaphoreType.DMA,
            [pltpu.SemaphoreType.DMA for _ in range(2)],
            [pltpu.SemaphoreType.DMA for _ in range(2)],
        ],
        compiler_params=pltpu.CompilerParams(use_tc_tiling_on_sc=False,
                                             needs_layout_passes=False),
    )
    def sad_sc(sites_hbm, c0_hbm, c1_hbm, par_hbm, out_hbm,
               table_v, c0_v, c1_v, out_v, par_v,
               tab_sem, in_sems, out_sems):
        wid = lax.axis_index("s") * N_CORES + lax.axis_index("c")
        base_w = wid * per_w

        def start_in(ci):
            b = ci % 2
            base = base_w + ci * CHUNK
            h0 = pltpu.async_copy(c0_hbm.at[pl.ds(base, CHUNK)], c0_v[b],
                                  in_sems[b])
            h1 = pltpu.async_copy(c1_hbm.at[pl.ds(base, CHUNK)], c1_v[b],
                                  in_sems[b])
            return (h0, h1)

        tab_h = pltpu.async_copy(sites_hbm, table_v, tab_sem)
        in_h = [None] * n_chunks
        out_h = [None] * n_chunks
        in_h[0] = start_in(0)
        pltpu.sync_copy(par_hbm, par_v)
        inv_w = par_v[pl.ds(0, L)]
        inv_h = par_v[pl.ds(L, L)]
        scale = par_v[pl.ds(2 * L, L)]
        iota = lax.iota(jnp.int32, L)
        tab_h.wait()

        for ci in range(n_chunks):
            b = ci % 2
            base = base_w + ci * CHUNK
            for h in in_h[ci]:
                h.wait()
            if ci + 1 < n_chunks:
                in_h[ci + 1] = start_in(ci + 1)
            if ci >= 2:
                for h in out_h[ci - 2]:
                    h.wait()
            c0b, c1b, outb = c0_v[b], c1_v[b], out_v[b]

            @plsc.parallel_loop(0, groups, 1, unroll=4)
            def grp(g):
                off = g * L
                w_off = base + off
                wo = w_off & (8 * width_s - 1)
                cc = lax.shift_right_logical(wo, 10)
                r = lax.shift_right_logical(wo, 7) & 7
                l0 = wo & 127
                t_glob = lax.shift_right_logical(w_off, 13)
                idx0 = c0b[pl.ds(off, L)]
                idx1 = c1b[pl.ds(off, L)]
                x0 = plsc.load_gather(table_v, [idx0])
                y0 = plsc.load_gather(table_v, [idx0 + n_sites])
                x1 = plsc.load_gather(table_v, [idx1])
                y1 = plsc.load_gather(table_v, [idx1 + n_sites])
                xi = cc * 128 + l0 + iota
                px = (xi.astype(jnp.float32) + 0.5) * inv_w
                yi = jnp.full((L,), t_glob * 8 + r, jnp.int32)
                py = (yi.astype(jnp.float32) + 0.5) * inv_h
                dx0 = px - x0
                dy0 = py - y0
                dx1 = px - x1
                dy1 = py - y1
                d0 = dx0 * dx0 + dy0 * dy0
                d1 = dx1 * dx1 + dy1 * dy1
                t = (d1 - d0) * scale
                w = 1.0 / (1.0 + jnp.exp(-t))
                for c in range(3):
                    a = plsc.load_gather(table_v, [idx0 + (2 + c) * n_sites])
                    bb = plsc.load_gather(table_v, [idx1 + (2 + c) * n_sites])
                    outb[c][pl.ds(off, L)] = bb + w * (a - bb)

            out_h[ci] = tuple(
                pltpu.async_copy(outb[c],
                                 out_hbm.at[pl.ds(c * npix + base, CHUNK)],
                                 out_sems[b])
                for c in range(3))

        for ci in (n_chunks - 2, n_chunks - 1):
            for h in out_h[ci]:
                h.wait()

    return sad_sc


def kernel(sites, cand0, cand1, width, height, inv_scale_sq):
    height_s, width_s = cand0.shape
    n_sites = sites.shape[0]
    npix = height_s * width_s

    width_f = jnp.asarray(width, dtype=jnp.float32)
    height_f = jnp.asarray(height, dtype=jnp.float32)
    scale_f = jnp.asarray(inv_scale_sq, dtype=jnp.float32)
    params = jnp.concatenate([
        jnp.broadcast_to(1.0 / width_f, (L,)),
        jnp.broadcast_to(1.0 / height_f, (L,)),
        jnp.broadcast_to(scale_f, (L,)),
    ]).astype(jnp.float32)

    def tile_order(c):
        c4 = c.reshape(height_s // 8, 8, width_s // 128, 128)
        return c4.transpose(0, 2, 1, 3).reshape(npix)

    sites_cols = sites.T.reshape(n_sites * 5)
    sad_sc = _build_sc_kernel(n_sites, npix, width_s)
    out_flat = sad_sc(sites_cols, tile_order(cand0), tile_order(cand1),
                      params)
    out5 = out_flat.reshape(3, height_s // 8, width_s // 128, 8, 128)
    return out5.transpose(1, 3, 2, 4, 0).reshape(height_s, width_s, 3)
